# Initial kernel scaffold; baseline (speedup 1.0000x reference)
import jax
import jax.numpy as jnp
from jax.experimental import pallas as pl


def kernel(x, edge_index, batch, l0_w1, l0_b1, l0_w2, l0_b2, l0_gamma, l0_beta, l1_w1, l1_b1, l1_w2, l1_b2, l1_gamma, l1_beta, l2_w1, l2_b1, l2_w2, l2_b2, l2_gamma, l2_beta, fc1_w, fc1_b, fc2_w, fc2_b):
    raise NotImplementedError("write your pallas kernel here")


# speedup vs baseline: 2.6353x; 2.6353x over previous
import functools

import jax
import jax.numpy as jnp
from jax import lax
from jax.experimental import pallas as pl
from jax.experimental.pallas import tpu as pltpu
from jax.experimental.pallas import tpu_sc as plsc

N = 10000
NPAD = 10240
E = 320000
D = 128
H = 128
G = 64

_INV = 0.9999950000374996

_EDGE_BLK = 128
_NTILES = 32
_BLK_PER_TILE = 80
_NBLOCKS = _NTILES * _BLK_PER_TILE
_EPAD = _NBLOCKS * _EDGE_BLK
_ROWS_PER_SUBCORE = NPAD // 16

_ROWS_BLK = 512
_TC_GRID = NPAD // _ROWS_BLK


def _sc_aggregate(h_pad, src2d, dst2d, zeros_blk):
    mesh = plsc.VectorSubcoreMesh(core_axis_name="c", subcore_axis_name="s")

    @functools.partial(
        pl.kernel,
        mesh=mesh,
        out_type=jax.ShapeDtypeStruct((2, NPAD, D), jnp.float32),
        scratch_types=[
            pltpu.VMEM((_BLK_PER_TILE, _EDGE_BLK), jnp.int32),
            pltpu.VMEM((_BLK_PER_TILE, _EDGE_BLK), jnp.int32),
            pltpu.VMEM((_EDGE_BLK, D), jnp.float32),
            pltpu.VMEM_SHARED((NPAD, D), jnp.float32),
            pltpu.SemaphoreType.DMA,
        ],
    )
    def k(h_hbm, src_hbm, dst_hbm, zeros_hbm, out_hbm,
          src_v, dst_v, rows_v, acc_sh, sem):
        c = lax.axis_index("c")
        s = lax.axis_index("s")
        wid = c * 16 + s

        pltpu.sync_copy(zeros_hbm, acc_sh.at[pl.ds(s * _ROWS_PER_SUBCORE,
                                                   _ROWS_PER_SUBCORE)])

        base_blk = wid * _BLK_PER_TILE
        pltpu.sync_copy(src_hbm.at[pl.ds(base_blk, _BLK_PER_TILE)], src_v)
        pltpu.sync_copy(dst_hbm.at[pl.ds(base_blk, _BLK_PER_TILE)], dst_v)

        plsc.subcore_barrier()

        def body(j, carry):
            pltpu.async_copy(h_hbm.at[src_v.at[j]], rows_v, sem).wait()
            pltpu.sync_copy(rows_v, acc_sh.at[dst_v.at[j]], add=True)
            return carry

        lax.fori_loop(0, _BLK_PER_TILE, body, 0)

        plsc.subcore_barrier()

        pltpu.sync_copy(
            acc_sh.at[pl.ds(s * _ROWS_PER_SUBCORE, _ROWS_PER_SUBCORE)],
            out_hbm.at[c, pl.ds(s * _ROWS_PER_SUBCORE, _ROWS_PER_SUBCORE)])

    return k(h_pad, src2d, dst2d, zeros_blk)


def _tc_layer(h, a0, a1, w1, b1, w2, b2, gamma, beta):
    def body(h_ref, a0_ref, a1_ref, w1_ref, b1_ref, w2_ref, b2_ref,
             g_ref, bt_ref, o_ref):
        hin = h_ref[...] + a0_ref[...] + a1_ref[...]
        z = jax.lax.dot_general(hin, w1_ref[...], (((1,), (0,)), ((), ())),
                                preferred_element_type=jnp.float32)
        z = jnp.maximum(z + b1_ref[...], 0.0)
        o = jax.lax.dot_general(z, w2_ref[...], (((1,), (0,)), ((), ())),
                                preferred_element_type=jnp.float32)
        o = (o + b2_ref[...]) * (g_ref[...] * _INV) + bt_ref[...]
        o_ref[...] = jnp.maximum(o, 0.0)

    row_spec = pl.BlockSpec((_ROWS_BLK, D), lambda i: (i, 0))
    full = lambda shape: pl.BlockSpec(shape, lambda i: (0,) * len(shape))
    return pl.pallas_call(
        body,
        grid=(_TC_GRID,),
        in_specs=[row_spec, row_spec, row_spec,
                  full((D, H)), full((1, H)), full((H, H)), full((1, H)),
                  full((1, H)), full((1, H))],
        out_specs=row_spec,
        out_shape=jax.ShapeDtypeStruct((NPAD, H), jnp.float32),
    )(h, a0, a1, w1, b1, w2, b2, gamma, beta)


def _tc_pool_head(h, batch3d, fc1_w, fc1_b, fc2_w, fc2_b):
    def body(h_ref, b_ref, w1_ref, b1_ref, w2_ref, b2_ref, o_ref, acc, cnt):
        i = pl.program_id(0)

        @pl.when(i == 0)
        def _():
            acc[...] = jnp.zeros_like(acc)
            cnt[...] = jnp.zeros_like(cnt)

        bm = b_ref[0]
        gids = jax.lax.broadcasted_iota(jnp.int32, (G, _ROWS_BLK), 0)
        onehot_t = (gids == bm).astype(jnp.float32)
        acc[...] += jax.lax.dot_general(
            onehot_t, h_ref[...], (((1,), (0,)), ((), ())),
            preferred_element_type=jnp.float32)
        cnt[...] += jnp.sum(onehot_t, axis=1, keepdims=True)

        @pl.when(i == _TC_GRID - 1)
        def _():
            pooled = acc[...] / jnp.maximum(cnt[...], 1.0)
            z = jax.lax.dot_general(pooled, w1_ref[...], (((1,), (0,)), ((), ())),
                                    preferred_element_type=jnp.float32)
            z = jnp.maximum(z + b1_ref[...], 0.0)
            o = jax.lax.dot_general(z, w2_ref[...], (((1,), (0,)), ((), ())),
                                    preferred_element_type=jnp.float32)
            o_ref[...] = o + b2_ref[...]

    full = lambda shape: pl.BlockSpec(shape, lambda i: (0,) * len(shape))
    return pl.pallas_call(
        body,
        grid=(_TC_GRID,),
        in_specs=[pl.BlockSpec((_ROWS_BLK, D), lambda i: (i, 0)),
                  pl.BlockSpec((1, 1, _ROWS_BLK), lambda i: (i, 0, 0)),
                  full((H, G)), full((1, G)), full((G, 1)), full((1, 1))],
        out_specs=full((G, 1)),
        out_shape=jax.ShapeDtypeStruct((G, 1), jnp.float32),
        scratch_shapes=[pltpu.VMEM((G, D), jnp.float32),
                        pltpu.VMEM((G, 1), jnp.float32)],
        compiler_params=pltpu.CompilerParams(
            dimension_semantics=("arbitrary",)),
    )(h, batch3d, fc1_w, fc1_b, fc2_w, fc2_b)


def kernel(x, edge_index, batch,
           l0_w1, l0_b1, l0_w2, l0_b2, l0_gamma, l0_beta,
           l1_w1, l1_b1, l1_w2, l1_b2, l1_gamma, l1_beta,
           l2_w1, l2_b1, l2_w2, l2_b2, l2_gamma, l2_beta,
           fc1_w, fc1_b, fc2_w, fc2_b):
    pad = jnp.full((_EPAD - E,), N, dtype=jnp.int32)
    src2d = jnp.concatenate([edge_index[0], pad]).reshape(_NBLOCKS, _EDGE_BLK)
    dst2d = jnp.concatenate([edge_index[1], pad]).reshape(_NBLOCKS, _EDGE_BLK)
    h = jnp.zeros((NPAD, D), jnp.float32).at[:N].set(x)
    batch3d = jnp.pad(batch, (0, NPAD - N), constant_values=G).reshape(
        _TC_GRID, 1, _ROWS_BLK)
    zeros_blk = jnp.zeros((_ROWS_PER_SUBCORE, D), jnp.float32)

    params = [
        (l0_w1, l0_b1, l0_w2, l0_b2, l0_gamma, l0_beta),
        (l1_w1, l1_b1, l1_w2, l1_b2, l1_gamma, l1_beta),
        (l2_w1, l2_b1, l2_w2, l2_b2, l2_gamma, l2_beta),
    ]
    for w1, b1, w2, b2, g, b in params:
        agg = _sc_aggregate(h, src2d, dst2d, zeros_blk)
        h = _tc_layer(h, agg[0], agg[1], w1, b1.reshape(1, H),
                      w2, b2.reshape(1, H), g.reshape(1, H), b.reshape(1, H))

    out = _tc_pool_head(h, batch3d, fc1_w, fc1_b.reshape(1, G),
                        fc2_w, fc2_b.reshape(1, 1))
    return jnp.squeeze(out, axis=-1)


# speedup vs baseline: 2.9609x; 1.1236x over previous
import functools

import jax
import jax.numpy as jnp
from jax import lax
from jax.experimental import pallas as pl
from jax.experimental.pallas import tpu as pltpu
from jax.experimental.pallas import tpu_sc as plsc

N = 10000
NPAD = 10240
E = 320000
D = 128
H = 128
G = 64

_INV = 0.9999950000374996

_EDGE_BLK = 128
_NTILES = 32
_BLK_PER_TILE = 80
_HALF = _BLK_PER_TILE // 2
_NBLOCKS = _NTILES * _BLK_PER_TILE
_EPAD = _NBLOCKS * _EDGE_BLK
_ROWS_PER_SUBCORE = NPAD // 16
_NBUF = 2

_ROWS_BLK = 512
_TC_GRID = NPAD // _ROWS_BLK


def _sc_aggregate(h_pad, src2d, dst2d, zeros_blk):
    mesh = plsc.VectorSubcoreMesh(core_axis_name="c", subcore_axis_name="s")

    @functools.partial(
        pl.kernel,
        mesh=mesh,
        out_type=jax.ShapeDtypeStruct((2, NPAD, D), jnp.float32),
        scratch_types=[
            pltpu.VMEM((_HALF, _EDGE_BLK), jnp.int32),
            pltpu.VMEM((_HALF, _EDGE_BLK), jnp.int32),
            pltpu.VMEM((_EDGE_BLK, D), jnp.float32),
            pltpu.VMEM((_EDGE_BLK, D), jnp.float32),
            pltpu.VMEM_SHARED((NPAD, D), jnp.float32),
            pltpu.SemaphoreType.DMA,
            pltpu.SemaphoreType.DMA,
        ],
    )
    def k(h_hbm, src_hbm, dst_hbm, zeros_hbm, out_hbm,
          src_v, dst_v, rows_v0, rows_v1, acc_sh, gsem0, gsem1):
        rows_b = (rows_v0, rows_v1)
        gsem_b = (gsem0, gsem1)
        c = lax.axis_index("c")
        s = lax.axis_index("s")
        wid = c * 16 + s

        pltpu.sync_copy(zeros_hbm, acc_sh.at[pl.ds(s * _ROWS_PER_SUBCORE,
                                                   _ROWS_PER_SUBCORE)])

        plsc.subcore_barrier()

        def start_gather(j, b):
            pltpu.async_copy(h_hbm.at[src_v.at[j]], rows_b[b], gsem_b[b])

        def wait_gather(j, b):
            pltpu.make_async_copy(h_hbm.at[src_v.at[j]], rows_b[b],
                                  gsem_b[b]).wait()

        base_blk = wid * _BLK_PER_TILE
        for half in range(2):
            hb = base_blk + half * _HALF
            pltpu.sync_copy(src_hbm.at[pl.ds(hb, _HALF)], src_v)
            pltpu.sync_copy(dst_hbm.at[pl.ds(hb, _HALF)], dst_v)

            for b in range(_NBUF):
                start_gather(b, b)

            def body(g, carry):
                for b in range(_NBUF):
                    j = g * _NBUF + b
                    wait_gather(j, b)
                    pltpu.sync_copy(rows_b[b], acc_sh.at[dst_v.at[j]],
                                    add=True)

                    @pl.when(j + _NBUF < _HALF)
                    def _():
                        start_gather(j + _NBUF, b)
                return carry

            lax.fori_loop(0, _HALF // _NBUF, body, 0)

        plsc.subcore_barrier()

        pltpu.sync_copy(
            acc_sh.at[pl.ds(s * _ROWS_PER_SUBCORE, _ROWS_PER_SUBCORE)],
            out_hbm.at[c, pl.ds(s * _ROWS_PER_SUBCORE, _ROWS_PER_SUBCORE)])

    return k(h_pad, src2d, dst2d, zeros_blk)


def _tc_layer(h, a0, a1, w1, b1, w2, b2, gamma, beta):
    def body(h_ref, a0_ref, a1_ref, w1_ref, b1_ref, w2_ref, b2_ref,
             g_ref, bt_ref, o_ref):
        hin = h_ref[...] + a0_ref[...] + a1_ref[...]
        z = jax.lax.dot_general(hin, w1_ref[...], (((1,), (0,)), ((), ())),
                                preferred_element_type=jnp.float32)
        z = jnp.maximum(z + b1_ref[...], 0.0)
        o = jax.lax.dot_general(z, w2_ref[...], (((1,), (0,)), ((), ())),
                                preferred_element_type=jnp.float32)
        o = (o + b2_ref[...]) * (g_ref[...] * _INV) + bt_ref[...]
        o_ref[...] = jnp.maximum(o, 0.0)

    row_spec = pl.BlockSpec((_ROWS_BLK, D), lambda i: (i, 0))
    full = lambda shape: pl.BlockSpec(shape, lambda i: (0,) * len(shape))
    return pl.pallas_call(
        body,
        grid=(_TC_GRID,),
        in_specs=[row_spec, row_spec, row_spec,
                  full((D, H)), full((1, H)), full((H, H)), full((1, H)),
                  full((1, H)), full((1, H))],
        out_specs=row_spec,
        out_shape=jax.ShapeDtypeStruct((NPAD, H), jnp.float32),
    )(h, a0, a1, w1, b1, w2, b2, gamma, beta)


def _tc_pool_head(h, batch3d, fc1_w, fc1_b, fc2_w, fc2_b):
    def body(h_ref, b_ref, w1_ref, b1_ref, w2_ref, b2_ref, o_ref, acc, cnt):
        i = pl.program_id(0)

        @pl.when(i == 0)
        def _():
            acc[...] = jnp.zeros_like(acc)
            cnt[...] = jnp.zeros_like(cnt)

        bm = b_ref[0]
        gids = jax.lax.broadcasted_iota(jnp.int32, (G, _ROWS_BLK), 0)
        onehot_t = (gids == bm).astype(jnp.float32)
        acc[...] += jax.lax.dot_general(
            onehot_t, h_ref[...], (((1,), (0,)), ((), ())),
            preferred_element_type=jnp.float32)
        cnt[...] += jnp.sum(onehot_t, axis=1, keepdims=True)

        @pl.when(i == _TC_GRID - 1)
        def _():
            pooled = acc[...] / jnp.maximum(cnt[...], 1.0)
            z = jax.lax.dot_general(pooled, w1_ref[...], (((1,), (0,)), ((), ())),
                                    preferred_element_type=jnp.float32)
            z = jnp.maximum(z + b1_ref[...], 0.0)
            o = jax.lax.dot_general(z, w2_ref[...], (((1,), (0,)), ((), ())),
                                    preferred_element_type=jnp.float32)
            o_ref[...] = o + b2_ref[...]

    full = lambda shape: pl.BlockSpec(shape, lambda i: (0,) * len(shape))
    return pl.pallas_call(
        body,
        grid=(_TC_GRID,),
        in_specs=[pl.BlockSpec((_ROWS_BLK, D), lambda i: (i, 0)),
                  pl.BlockSpec((1, 1, _ROWS_BLK), lambda i: (i, 0, 0)),
                  full((H, G)), full((1, G)), full((G, 1)), full((1, 1))],
        out_specs=full((G, 1)),
        out_shape=jax.ShapeDtypeStruct((G, 1), jnp.float32),
        scratch_shapes=[pltpu.VMEM((G, D), jnp.float32),
                        pltpu.VMEM((G, 1), jnp.float32)],
        compiler_params=pltpu.CompilerParams(
            dimension_semantics=("arbitrary",)),
    )(h, batch3d, fc1_w, fc1_b, fc2_w, fc2_b)


def kernel(x, edge_index, batch,
           l0_w1, l0_b1, l0_w2, l0_b2, l0_gamma, l0_beta,
           l1_w1, l1_b1, l1_w2, l1_b2, l1_gamma, l1_beta,
           l2_w1, l2_b1, l2_w2, l2_b2, l2_gamma, l2_beta,
           fc1_w, fc1_b, fc2_w, fc2_b):
    pad = jnp.full((_EPAD - E,), N, dtype=jnp.int32)
    src2d = jnp.concatenate([edge_index[0], pad]).reshape(_NBLOCKS, _EDGE_BLK)
    dst2d = jnp.concatenate([edge_index[1], pad]).reshape(_NBLOCKS, _EDGE_BLK)
    h = jnp.zeros((NPAD, D), jnp.float32).at[:N].set(x)
    batch3d = jnp.pad(batch, (0, NPAD - N), constant_values=G).reshape(
        _TC_GRID, 1, _ROWS_BLK)
    zeros_blk = jnp.zeros((_ROWS_PER_SUBCORE, D), jnp.float32)

    params = [
        (l0_w1, l0_b1, l0_w2, l0_b2, l0_gamma, l0_beta),
        (l1_w1, l1_b1, l1_w2, l1_b2, l1_gamma, l1_beta),
        (l2_w1, l2_b1, l2_w2, l2_b2, l2_gamma, l2_beta),
    ]
    for w1, b1, w2, b2, g, b in params:
        agg = _sc_aggregate(h, src2d, dst2d, zeros_blk)
        h = _tc_layer(h, agg[0], agg[1], w1, b1.reshape(1, H),
                      w2, b2.reshape(1, H), g.reshape(1, H), b.reshape(1, H))

    out = _tc_pool_head(h, batch3d, fc1_w, fc1_b.reshape(1, G),
                        fc2_w, fc2_b.reshape(1, 1))
    return jnp.squeeze(out, axis=-1)


# speedup vs baseline: 9.9913x; 3.3744x over previous
import functools

import jax
import jax.numpy as jnp
from jax import lax
from jax.experimental import pallas as pl
from jax.experimental.pallas import tpu as pltpu
from jax.experimental.pallas import tpu_sc as plsc

N = 10000
NPAD = 10240
E = 320000
D = 128
H = 128
G = 64

_INV = 0.9999950000374996

_EDGE_BLK = 128
_NTILES = 32
_BLK_PER_TILE = 80
_HALF = _BLK_PER_TILE // 2
_NBLOCKS = _NTILES * _BLK_PER_TILE
_EPAD = _NBLOCKS * _EDGE_BLK
_ROWS_PER_SUBCORE = NPAD // 16
_NBUF = 2

_ROWS_BLK = 512
_TC_GRID = NPAD // _ROWS_BLK


def _sc_aggregate(h_pad, src2d, dst2d, zeros_blk):
    mesh = plsc.VectorSubcoreMesh(core_axis_name="c", subcore_axis_name="s")

    @functools.partial(
        pl.kernel,
        mesh=mesh,
        out_type=jax.ShapeDtypeStruct((2, NPAD, D), jnp.float32),
        scratch_types=[
            pltpu.VMEM((_HALF, _EDGE_BLK), jnp.int32),
            pltpu.VMEM((_HALF, _EDGE_BLK), jnp.int32),
            pltpu.VMEM((_EDGE_BLK, D), jnp.float32),
            pltpu.VMEM((_EDGE_BLK, D), jnp.float32),
            pltpu.VMEM_SHARED((NPAD, D), jnp.float32),
            pltpu.SemaphoreType.DMA,
            pltpu.SemaphoreType.DMA,
        ],
    )
    def k(h_hbm, src_hbm, dst_hbm, zeros_hbm, out_hbm,
          src_v, dst_v, rows_v0, rows_v1, acc_sh, gsem0, gsem1):
        rows_b = (rows_v0, rows_v1)
        gsem_b = (gsem0, gsem1)
        c = lax.axis_index("c")
        s = lax.axis_index("s")
        wid = c * 16 + s

        pltpu.sync_copy(zeros_hbm, acc_sh.at[pl.ds(s * _ROWS_PER_SUBCORE,
                                                   _ROWS_PER_SUBCORE)])

        plsc.subcore_barrier()

        def start_gather(j, b):
            pltpu.async_copy(h_hbm.at[src_v.at[j]], rows_b[b], gsem_b[b])

        def wait_gather(j, b):
            pltpu.make_async_copy(h_hbm.at[src_v.at[j]], rows_b[b],
                                  gsem_b[b]).wait()

        base_blk = wid * _BLK_PER_TILE
        for half in range(2):
            hb = base_blk + half * _HALF
            pltpu.sync_copy(src_hbm.at[pl.ds(hb, _HALF)], src_v)
            pltpu.sync_copy(dst_hbm.at[pl.ds(hb, _HALF)], dst_v)

            for b in range(_NBUF):
                start_gather(b, b)

            def body(g, carry):
                for b in range(_NBUF):
                    j = g * _NBUF + b
                    wait_gather(j, b)
                    pltpu.sync_copy(rows_b[b], acc_sh.at[dst_v.at[j]],
                                    add=True)

                    @pl.when(j + _NBUF < _HALF)
                    def _():
                        start_gather(j + _NBUF, b)
                return carry

            lax.fori_loop(0, _HALF // _NBUF, body, 0)

        plsc.subcore_barrier()

        pltpu.sync_copy(
            acc_sh.at[pl.ds(s * _ROWS_PER_SUBCORE, _ROWS_PER_SUBCORE)],
            out_hbm.at[c, pl.ds(s * _ROWS_PER_SUBCORE, _ROWS_PER_SUBCORE)])

    return k(h_pad, src2d, dst2d, zeros_blk)


def _tc_layer(h, a0, a1, w1, b1, w2, b2, gamma, beta):
    def body(h_ref, a0_ref, a1_ref, w1_ref, b1_ref, w2_ref, b2_ref,
             g_ref, bt_ref, o_ref):
        hin = h_ref[...] + a0_ref[...] + a1_ref[...]
        z = jax.lax.dot_general(hin, w1_ref[...], (((1,), (0,)), ((), ())),
                                preferred_element_type=jnp.float32)
        z = jnp.maximum(z + b1_ref[...], 0.0)
        o = jax.lax.dot_general(z, w2_ref[...], (((1,), (0,)), ((), ())),
                                preferred_element_type=jnp.float32)
        o = (o + b2_ref[...]) * (g_ref[...] * _INV) + bt_ref[...]
        o_ref[...] = jnp.maximum(o, 0.0)

    row_spec = pl.BlockSpec((_ROWS_BLK, D), lambda i: (i, 0))
    full = lambda shape: pl.BlockSpec(shape, lambda i: (0,) * len(shape))
    return pl.pallas_call(
        body,
        grid=(_TC_GRID,),
        in_specs=[row_spec, row_spec, row_spec,
                  full((D, H)), full((1, H)), full((H, H)), full((1, H)),
                  full((1, H)), full((1, H))],
        out_specs=row_spec,
        out_shape=jax.ShapeDtypeStruct((NPAD, H), jnp.float32),
    )(h, a0, a1, w1, b1, w2, b2, gamma, beta)


def _tc_pool_head(h, batch3d, fc1_w, fc1_b, fc2_w, fc2_b):
    def body(h_ref, b_ref, w1_ref, b1_ref, w2_ref, b2_ref, o_ref, acc, cnt):
        i = pl.program_id(0)

        @pl.when(i == 0)
        def _():
            acc[...] = jnp.zeros_like(acc)
            cnt[...] = jnp.zeros_like(cnt)

        bm = b_ref[0]
        gids = jax.lax.broadcasted_iota(jnp.int32, (G, _ROWS_BLK), 0)
        onehot_t = (gids == bm).astype(jnp.float32)
        acc[...] += jax.lax.dot_general(
            onehot_t, h_ref[...], (((1,), (0,)), ((), ())),
            preferred_element_type=jnp.float32)
        cnt[...] += jnp.sum(onehot_t, axis=1, keepdims=True)

        @pl.when(i == _TC_GRID - 1)
        def _():
            pooled = acc[...] / jnp.maximum(cnt[...], 1.0)
            z = jax.lax.dot_general(pooled, w1_ref[...], (((1,), (0,)), ((), ())),
                                    preferred_element_type=jnp.float32)
            z = jnp.maximum(z + b1_ref[...], 0.0)
            o = jax.lax.dot_general(z, w2_ref[...], (((1,), (0,)), ((), ())),
                                    preferred_element_type=jnp.float32)
            o_ref[...] = o + b2_ref[...]

    full = lambda shape: pl.BlockSpec(shape, lambda i: (0,) * len(shape))
    return pl.pallas_call(
        body,
        grid=(_TC_GRID,),
        in_specs=[pl.BlockSpec((_ROWS_BLK, D), lambda i: (i, 0)),
                  pl.BlockSpec((1, 1, _ROWS_BLK), lambda i: (i, 0, 0)),
                  full((H, G)), full((1, G)), full((G, 1)), full((1, 1))],
        out_specs=full((G, 1)),
        out_shape=jax.ShapeDtypeStruct((G, 1), jnp.float32),
        scratch_shapes=[pltpu.VMEM((G, D), jnp.float32),
                        pltpu.VMEM((G, 1), jnp.float32)],
        compiler_params=pltpu.CompilerParams(
            dimension_semantics=("arbitrary",)),
    )(h, batch3d, fc1_w, fc1_b, fc2_w, fc2_b)


def kernel(x, edge_index, batch,
           l0_w1, l0_b1, l0_w2, l0_b2, l0_gamma, l0_beta,
           l1_w1, l1_b1, l1_w2, l1_b2, l1_gamma, l1_beta,
           l2_w1, l2_b1, l2_w2, l2_b2, l2_gamma, l2_beta,
           fc1_w, fc1_b, fc2_w, fc2_b):
    pad = N + (jnp.arange(_EPAD - E, dtype=jnp.int32) % (NPAD - N))
    src2d = jnp.concatenate([edge_index[0], pad]).reshape(_NBLOCKS, _EDGE_BLK)
    dst2d = jnp.concatenate([edge_index[1], pad]).reshape(_NBLOCKS, _EDGE_BLK)
    h = jnp.zeros((NPAD, D), jnp.float32).at[:N].set(x)
    batch3d = jnp.pad(batch, (0, NPAD - N), constant_values=G).reshape(
        _TC_GRID, 1, _ROWS_BLK)
    zeros_blk = jnp.zeros((_ROWS_PER_SUBCORE, D), jnp.float32)

    params = [
        (l0_w1, l0_b1, l0_w2, l0_b2, l0_gamma, l0_beta),
        (l1_w1, l1_b1, l1_w2, l1_b2, l1_gamma, l1_beta),
        (l2_w1, l2_b1, l2_w2, l2_b2, l2_gamma, l2_beta),
    ]
    for w1, b1, w2, b2, g, b in params:
        agg = _sc_aggregate(h, src2d, dst2d, zeros_blk)
        h = _tc_layer(h, agg[0], agg[1], w1, b1.reshape(1, H),
                      w2, b2.reshape(1, H), g.reshape(1, H), b.reshape(1, H))

    out = _tc_pool_head(h, batch3d, fc1_w, fc1_b.reshape(1, G),
                        fc2_w, fc2_b.reshape(1, 1))
    return jnp.squeeze(out, axis=-1)


# speedup vs baseline: 10.4805x; 1.0490x over previous
import functools

import jax
import jax.numpy as jnp
from jax import lax
from jax.experimental import pallas as pl
from jax.experimental.pallas import tpu as pltpu
from jax.experimental.pallas import tpu_sc as plsc

N = 10000
NPAD = 10240
E = 320000
D = 128
H = 128
G = 64

_INV = 0.9999950000374996

_EDGE_BLK = 128
_NTILES = 32
_BLK_PER_TILE = 80
_HALF = _BLK_PER_TILE // 2
_NBLOCKS = _NTILES * _BLK_PER_TILE
_EPAD = _NBLOCKS * _EDGE_BLK
_ROWS_PER_SUBCORE = NPAD // 16
_NBUF = 2

_ROWS_BLK = 512
_TC_GRID = NPAD // _ROWS_BLK


def _sc_aggregate(h_pad, src2d, dst2d, zeros_blk):
    mesh = plsc.VectorSubcoreMesh(core_axis_name="c", subcore_axis_name="s")

    @functools.partial(
        pl.kernel,
        mesh=mesh,
        out_type=(jax.ShapeDtypeStruct((NPAD, D), jnp.float32),
                  jax.ShapeDtypeStruct((NPAD, D), jnp.float32)),
        scratch_types=[
            pltpu.VMEM((_HALF, _EDGE_BLK), jnp.int32),
            pltpu.VMEM((_HALF, _EDGE_BLK), jnp.int32),
            pltpu.VMEM((_EDGE_BLK, D), jnp.float32),
            pltpu.VMEM((_EDGE_BLK, D), jnp.float32),
            pltpu.VMEM_SHARED((NPAD, D), jnp.float32),
            pltpu.SemaphoreType.DMA,
            pltpu.SemaphoreType.DMA,
        ],
    )
    def k(h_hbm, src_hbm, dst_hbm, zeros_hbm, out0_hbm, out1_hbm,
          src_v, dst_v, rows_v0, rows_v1, acc_sh, gsem0, gsem1):
        rows_b = (rows_v0, rows_v1)
        gsem_b = (gsem0, gsem1)
        c = lax.axis_index("c")
        s = lax.axis_index("s")
        wid = c * 16 + s

        pltpu.sync_copy(zeros_hbm, acc_sh.at[pl.ds(s * _ROWS_PER_SUBCORE,
                                                   _ROWS_PER_SUBCORE)])

        plsc.subcore_barrier()

        def start_gather(j, b):
            pltpu.async_copy(h_hbm.at[src_v.at[j]], rows_b[b], gsem_b[b])

        def wait_gather(j, b):
            pltpu.make_async_copy(h_hbm.at[src_v.at[j]], rows_b[b],
                                  gsem_b[b]).wait()

        base_blk = wid * _BLK_PER_TILE
        for half in range(2):
            hb = base_blk + half * _HALF
            pltpu.sync_copy(src_hbm.at[pl.ds(hb, _HALF)], src_v)
            pltpu.sync_copy(dst_hbm.at[pl.ds(hb, _HALF)], dst_v)

            for b in range(_NBUF):
                start_gather(b, b)

            def body(g, carry):
                for b in range(_NBUF):
                    j = g * _NBUF + b
                    wait_gather(j, b)
                    pltpu.sync_copy(rows_b[b], acc_sh.at[dst_v.at[j]],
                                    add=True)

                    @pl.when(j + _NBUF < _HALF)
                    def _():
                        start_gather(j + _NBUF, b)
                return carry

            lax.fori_loop(0, _HALF // _NBUF, body, 0)

        plsc.subcore_barrier()

        my_rows = pl.ds(s * _ROWS_PER_SUBCORE, _ROWS_PER_SUBCORE)

        @pl.when(c == 0)
        def _():
            pltpu.sync_copy(acc_sh.at[my_rows], out0_hbm.at[my_rows])

        @pl.when(c == 1)
        def _():
            pltpu.sync_copy(acc_sh.at[my_rows], out1_hbm.at[my_rows])

    return k(h_pad, src2d, dst2d, zeros_blk)


def _tc_layer(h, a0, a1, w1, b1, w2, b2, gamma, beta):
    def body(h_ref, a0_ref, a1_ref, w1_ref, b1_ref, w2_ref, b2_ref,
             g_ref, bt_ref, o_ref):
        hin = h_ref[...] + a0_ref[...] + a1_ref[...]
        z = jax.lax.dot_general(hin, w1_ref[...], (((1,), (0,)), ((), ())),
                                preferred_element_type=jnp.float32)
        z = jnp.maximum(z + b1_ref[...], 0.0)
        o = jax.lax.dot_general(z, w2_ref[...], (((1,), (0,)), ((), ())),
                                preferred_element_type=jnp.float32)
        o = (o + b2_ref[...]) * (g_ref[...] * _INV) + bt_ref[...]
        o_ref[...] = jnp.maximum(o, 0.0)

    row_spec = pl.BlockSpec((_ROWS_BLK, D), lambda i: (i, 0))
    full = lambda shape: pl.BlockSpec(shape, lambda i: (0,) * len(shape))
    return pl.pallas_call(
        body,
        grid=(_TC_GRID,),
        in_specs=[row_spec, row_spec, row_spec,
                  full((D, H)), full((1, H)), full((H, H)), full((1, H)),
                  full((1, H)), full((1, H))],
        out_specs=row_spec,
        out_shape=jax.ShapeDtypeStruct((NPAD, H), jnp.float32),
    )(h, a0, a1, w1, b1, w2, b2, gamma, beta)


def _tc_pool_head(h, batch3d, fc1_w, fc1_b, fc2_w, fc2_b):
    def body(h_ref, b_ref, w1_ref, b1_ref, w2_ref, b2_ref, o_ref, acc, cnt):
        i = pl.program_id(0)

        @pl.when(i == 0)
        def _():
            acc[...] = jnp.zeros_like(acc)
            cnt[...] = jnp.zeros_like(cnt)

        bm = b_ref[0]
        gids = jax.lax.broadcasted_iota(jnp.int32, (G, _ROWS_BLK), 0)
        onehot_t = (gids == bm).astype(jnp.float32)
        acc[...] += jax.lax.dot_general(
            onehot_t, h_ref[...], (((1,), (0,)), ((), ())),
            preferred_element_type=jnp.float32)
        cnt[...] += jnp.sum(onehot_t, axis=1, keepdims=True)

        @pl.when(i == _TC_GRID - 1)
        def _():
            pooled = acc[...] / jnp.maximum(cnt[...], 1.0)
            z = jax.lax.dot_general(pooled, w1_ref[...], (((1,), (0,)), ((), ())),
                                    preferred_element_type=jnp.float32)
            z = jnp.maximum(z + b1_ref[...], 0.0)
            o = jax.lax.dot_general(z, w2_ref[...], (((1,), (0,)), ((), ())),
                                    preferred_element_type=jnp.float32)
            o_ref[...] = o + b2_ref[...]

    full = lambda shape: pl.BlockSpec(shape, lambda i: (0,) * len(shape))
    return pl.pallas_call(
        body,
        grid=(_TC_GRID,),
        in_specs=[pl.BlockSpec((_ROWS_BLK, D), lambda i: (i, 0)),
                  pl.BlockSpec((1, 1, _ROWS_BLK), lambda i: (i, 0, 0)),
                  full((H, G)), full((1, G)), full((G, 1)), full((1, 1))],
        out_specs=full((G, 1)),
        out_shape=jax.ShapeDtypeStruct((G, 1), jnp.float32),
        scratch_shapes=[pltpu.VMEM((G, D), jnp.float32),
                        pltpu.VMEM((G, 1), jnp.float32)],
        compiler_params=pltpu.CompilerParams(
            dimension_semantics=("arbitrary",)),
    )(h, batch3d, fc1_w, fc1_b, fc2_w, fc2_b)


def kernel(x, edge_index, batch,
           l0_w1, l0_b1, l0_w2, l0_b2, l0_gamma, l0_beta,
           l1_w1, l1_b1, l1_w2, l1_b2, l1_gamma, l1_beta,
           l2_w1, l2_b1, l2_w2, l2_b2, l2_gamma, l2_beta,
           fc1_w, fc1_b, fc2_w, fc2_b):
    pad = N + (jnp.arange(_EPAD - E, dtype=jnp.int32) % (NPAD - N))
    src2d = jnp.concatenate([edge_index[0], pad]).reshape(_NBLOCKS, _EDGE_BLK)
    dst2d = jnp.concatenate([edge_index[1], pad]).reshape(_NBLOCKS, _EDGE_BLK)
    h = jnp.zeros((NPAD, D), jnp.float32).at[:N].set(x)
    batch3d = jnp.pad(batch, (0, NPAD - N), constant_values=G).reshape(
        _TC_GRID, 1, _ROWS_BLK)
    zeros_blk = jnp.zeros((_ROWS_PER_SUBCORE, D), jnp.float32)

    params = [
        (l0_w1, l0_b1, l0_w2, l0_b2, l0_gamma, l0_beta),
        (l1_w1, l1_b1, l1_w2, l1_b2, l1_gamma, l1_beta),
        (l2_w1, l2_b1, l2_w2, l2_b2, l2_gamma, l2_beta),
    ]
    for w1, b1, w2, b2, g, b in params:
        agg0, agg1 = _sc_aggregate(h, src2d, dst2d, zeros_blk)
        h = _tc_layer(h, agg0, agg1, w1, b1.reshape(1, H),
                      w2, b2.reshape(1, H), g.reshape(1, H), b.reshape(1, H))

    out = _tc_pool_head(h, batch3d, fc1_w, fc1_b.reshape(1, G),
                        fc2_w, fc2_b.reshape(1, 1))
    return jnp.squeeze(out, axis=-1)
